# baseline (speedup 1.0000x reference)
import jax
import jax.numpy as jnp
from jax import lax
from jax.experimental import pallas as pl


_BR = 64


def _body(steps_ref, tab_ref, out_ref):
    steps = steps_ref[0, 0, :]
    tab = tab_ref[0, :]
    t = tab.shape[0]
    br = steps.shape[0]
    lane = lax.broadcasted_iota(jnp.int32, (br, t), 1)
    eq = lane == steps[:, None]
    vals = jnp.sum(jnp.where(eq, tab[None, :], 0.0), axis=1)
    out_ref[...] = jnp.broadcast_to(vals[:, None], out_ref.shape)


def kernel(img, num_steps, alpha_bars):
    b, c, h, w = img.shape
    f = c * h * w
    t_pad = 1024
    tab = jnp.zeros((1, t_pad), jnp.float32).at[0, : alpha_bars.shape[0]].set(
        alpha_bars
    )
    g = b // _BR
    steps3d = num_steps.reshape(g, 1, _BR)

    out = pl.pallas_call(
        _body,
        grid=(g,),
        in_specs=[
            pl.BlockSpec((1, 1, _BR), lambda i: (i, 0, 0)),
            pl.BlockSpec((1, t_pad), lambda i: (0, 0)),
        ],
        out_specs=pl.BlockSpec((_BR, f), lambda i: (i, 0)),
        out_shape=jax.ShapeDtypeStruct((b, f), jnp.float32),
    )(steps3d, tab)
    return out.reshape(b, c, h, w)


# speedup vs baseline: 1.0274x; 1.0274x over previous
import jax
import jax.numpy as jnp
from jax import lax
from jax.experimental import pallas as pl


_BR = 256


def _body(steps_ref, tab_ref, out_ref):
    steps = steps_ref[0, 0, :]
    tab = tab_ref[0, :]
    t = tab.shape[0]
    br = steps.shape[0]
    lane = lax.broadcasted_iota(jnp.int32, (br, t), 1)
    eq = lane == steps[:, None]
    vals = jnp.sum(jnp.where(eq, tab[None, :], 0.0), axis=1)
    out_ref[...] = jnp.broadcast_to(vals[:, None], out_ref.shape)


def kernel(img, num_steps, alpha_bars):
    b, c, h, w = img.shape
    f = c * h * w
    t_pad = 1024
    tab = jnp.zeros((1, t_pad), jnp.float32).at[0, : alpha_bars.shape[0]].set(
        alpha_bars
    )
    g = b // _BR
    steps3d = num_steps.reshape(g, 1, _BR)

    out = pl.pallas_call(
        _body,
        grid=(g,),
        in_specs=[
            pl.BlockSpec((1, 1, _BR), lambda i: (i, 0, 0)),
            pl.BlockSpec((1, t_pad), lambda i: (0, 0)),
        ],
        out_specs=pl.BlockSpec((_BR, f), lambda i: (i, 0)),
        out_shape=jax.ShapeDtypeStruct((b, f), jnp.float32),
    )(steps3d, tab)
    return out.reshape(b, c, h, w)
